# baseline (speedup 1.0000x reference)
import functools

import jax
import jax.numpy as jnp
from jax import lax
from jax.experimental import pallas as pl
from jax.experimental.pallas import tpu as pltpu
from jax.experimental.pallas import tpu_sc as plsc

N = 100000
D = 512
G = 256
NC = 2
NS = 16
L = 16
NW = NC * NS
NJ = D // L
DROW = D + L

CH = 2 * L
NCHUNK = N // L
BIGW = (NCHUNK // 2) % NW
CPW = NCHUNK // 2 // NW
BIG_ROWS = (CPW + 1) * CH
SMALL_ROWS = CPW * CH
BIG_BASE = BIGW * BIG_ROWS

SEG_PER_W = G // NW

_SPLAT_DNUMS = lax.GatherDimensionNumbers(
    offset_dims=(), collapsed_slice_dims=(0,), start_index_map=(0,))


def _vgather(vec, idx):
    return lax.gather(vec, idx.reshape(L, 1), _SPLAT_DNUMS, slice_sizes=(1,),
                      mode=lax.GatherScatterMode.PROMISE_IN_BOUNDS)


def _splat(vec, lane):
    return _vgather(vec, jnp.full((L,), lane, jnp.int32))


_mesh = functools.partial(
    plsc.VectorSubcoreMesh,
    core_axis_name="c", subcore_axis_name="s", num_cores=NC, num_subcores=NS)

_params = pltpu.CompilerParams(needs_layout_passes=False)


@functools.partial(
    pl.kernel,
    out_type=[
        jax.ShapeDtypeStruct((G * D,), jnp.float32),
        jax.ShapeDtypeStruct((NW * 2 * DROW,), jnp.float32),
        jax.ShapeDtypeStruct((NW * L,), jnp.int32),
    ],
    mesh=_mesh(),
    compiler_params=_params,
    scratch_types=[
        pltpu.VMEM((CH, D), jnp.float32),
        pltpu.VMEM((CH, D), jnp.float32),
        pltpu.VMEM((BIG_ROWS,), jnp.float32),
        pltpu.VMEM((BIG_ROWS + L,), jnp.int32),
        pltpu.VMEM((L,), jnp.int32),
        pltpu.VMEM((L,), jnp.int32),
        pltpu.VMEM((D,), jnp.float32),
        pltpu.VMEM((L,), jnp.float32),
        pltpu.VMEM((D,), jnp.float32),
        pltpu.VMEM((L,), jnp.int32),
        pltpu.SemaphoreType.DMA,
        pltpu.SemaphoreType.DMA,
    ],
)
def _pool(attr_h, att_h, bi_h, out1, slab_out, meta_out,
          rows_a, rows_b, att_v, bi_v, ptail_v, nhead_v, acc_v, accw_v, zrow_v,
          meta_v, sem_a, sem_b):
    cid = lax.axis_index("c")
    sid = lax.axis_index("s")
    wid = sid * NC + cid

    nch = jnp.where(wid < BIGW, CPW + 1, CPW)
    nrows = nch * CH
    wbase = jnp.where(wid < BIGW, wid * BIG_ROWS,
                      BIG_BASE + (wid - BIGW) * SMALL_ROWS)

    zero = jnp.zeros((L,), jnp.float32)
    for j in range(NJ):
        acc_v[pl.ds(j * L, L)] = zero
        zrow_v[pl.ds(j * L, L)] = zero
    accw_v[...] = zero

    ptail_v[...] = jnp.full((L,), -1, jnp.int32)
    nhead_v[...] = jnp.full((L,), -1, jnp.int32)

    @pl.when(wid > 0)
    def _():
        pltpu.sync_copy(bi_h.at[pl.ds(wbase - L, L)], ptail_v)

    @pl.when(wid < NW - 1)
    def _():
        pltpu.sync_copy(bi_h.at[pl.ds(wbase + nrows, L)], nhead_v)

    @pl.when(wid < BIGW)
    def _():
        pltpu.sync_copy(att_h.at[pl.ds(wbase, BIG_ROWS)],
                        att_v.at[pl.ds(0, BIG_ROWS)])
        pltpu.sync_copy(bi_h.at[pl.ds(wbase, BIG_ROWS)],
                        bi_v.at[pl.ds(0, BIG_ROWS)])

    @pl.when(wid >= BIGW)
    def _():
        pltpu.sync_copy(att_h.at[pl.ds(wbase, SMALL_ROWS)],
                        att_v.at[pl.ds(0, SMALL_ROWS)])
        pltpu.sync_copy(bi_h.at[pl.ds(wbase, SMALL_ROWS)],
                        bi_v.at[pl.ds(0, SMALL_ROWS)])

    first_seg = bi_v[pl.ds(0, L)][0]
    last_seg = bi_v[pl.ds(nrows - L, L)][L - 1]
    first_shared = ptail_v[pl.ds(0, L)][L - 1] == first_seg
    last_shared = nhead_v[pl.ds(0, L)][0] == last_seg

    def _flush(seg):
        fs = (seg == first_seg) & first_shared
        ls = (seg == last_seg) & last_shared

        @pl.when(fs)
        def _():
            pltpu.sync_copy(acc_v, slab_out.at[pl.ds(wid * 2 * DROW, D)])
            pltpu.sync_copy(accw_v, slab_out.at[pl.ds(wid * 2 * DROW + D, L)])

        @pl.when(ls)
        def _():
            pltpu.sync_copy(acc_v,
                            slab_out.at[pl.ds((wid * 2 + 1) * DROW, D)])
            pltpu.sync_copy(accw_v,
                            slab_out.at[pl.ds((wid * 2 + 1) * DROW + D, L)])

        @pl.when(jnp.logical_not(fs | ls))
        def _():
            scale = 1.0 / accw_v[pl.ds(0, L)]
            for j in range(NJ):
                acc_v[pl.ds(j * L, L)] = acc_v[pl.ds(j * L, L)] * scale
            pltpu.sync_copy(acc_v, out1.at[pl.ds(seg * D, D)])

        for j in range(NJ):
            acc_v[pl.ds(j * L, L)] = zero
        accw_v[...] = zero

    lanes = lax.iota(jnp.int32, L)
    shift_idx = jnp.maximum(lanes - 1, 0)

    def _process(buf, rbase, lbase, prev):
        biv = bi_v[pl.ds(lbase, L)]
        wv = jnp.exp(att_v[pl.ds(lbase, L)])
        shifted = _vgather(biv, shift_idx)
        shifted = jnp.where(lanes == 0, jnp.full((L,), prev, jnp.int32), shifted)
        has_boundary = jnp.any(shifted != biv)

        @pl.when(jnp.logical_not(has_boundary))
        def _():
            wsum = _splat(wv, 0)
            for r in range(1, L):
                wsum = wsum + _splat(wv, r)
            plsc.addupdate(accw_v.at[pl.ds(0, L)], wsum)

            def tranche(t, _):
                jb = t * 8 * L
                w = _splat(wv, 0)
                regs = [buf[rbase, pl.ds(jb + j * L, L)] * w for j in range(8)]
                for r in range(1, L):
                    w = _splat(wv, r)
                    for j in range(8):
                        regs[j] = regs[j] + buf[rbase + r, pl.ds(jb + j * L, L)] * w
                for j in range(8):
                    plsc.addupdate(acc_v.at[pl.ds(jb + j * L, L)], regs[j])
                return 0
            lax.fori_loop(0, NJ // 8, tranche, 0)

        @pl.when(has_boundary)
        def _():
            def lane_body(li, prevl):
                s = bi_v[pl.ds(lbase + li, L)][0]

                @pl.when(s != prevl)
                def _():
                    _flush(prevl)

                    def gap_body(z):
                        pltpu.sync_copy(zrow_v, out1.at[pl.ds(z * D, D)])
                        return z + 1
                    lax.while_loop(lambda z: z < s, gap_body, prevl + 1)

                w = _vgather(wv, jnp.full((L,), li, jnp.int32))
                plsc.addupdate(accw_v.at[pl.ds(0, L)], w)
                for j in range(NJ):
                    plsc.addupdate(acc_v.at[pl.ds(j * L, L)],
                                   buf[rbase + li, pl.ds(j * L, L)] * w)
                return s
            lax.fori_loop(0, L, lane_body, prev)

        return biv[L - 1]

    pltpu.async_copy(attr_h.at[pl.ds(wbase, CH)], rows_a, sem_a)
    pltpu.async_copy(attr_h.at[pl.ds(wbase + CH, CH)], rows_b, sem_b)

    def _chunk(buf, c, prev):
        for gi in range(CH // L):
            prev = _process(buf, gi * L, c * CH + gi * L, prev)
        return prev

    def _pair(p, prev):
        c0 = 2 * p
        base0 = wbase + c0 * CH
        base1 = base0 + CH
        pltpu.make_async_copy(attr_h.at[pl.ds(base0, CH)], rows_a, sem_a).wait()
        prev = _chunk(rows_a, c0, prev)

        @pl.when(c0 + 2 < nch)
        def _():
            pltpu.async_copy(attr_h.at[pl.ds(base0 + 2 * CH, CH)], rows_a,
                             sem_a)

        pltpu.make_async_copy(attr_h.at[pl.ds(base1, CH)], rows_b, sem_b).wait()
        prev = _chunk(rows_b, c0 + 1, prev)

        @pl.when(c0 + 3 < nch)
        def _():
            pltpu.async_copy(attr_h.at[pl.ds(base1 + 2 * CH, CH)], rows_b,
                             sem_b)

        return prev

    prev = lax.fori_loop(0, nch // 2, _pair, first_seg)

    @pl.when(nch % 2 == 1)
    def _():
        c_t = nch - 1
        pltpu.make_async_copy(attr_h.at[pl.ds(wbase + c_t * CH, CH)], rows_a,
                              sem_a).wait()
        prev_t = _chunk(rows_a, c_t, prev)
        _flush(prev_t)

    @pl.when(nch % 2 == 0)
    def _():
        _flush(prev)

    fs_code = jnp.where(first_shared, first_seg, -1)
    ls_code = jnp.where(last_shared, last_seg, -1)
    mrow = jnp.full((L,), -1, jnp.int32)
    mrow = jnp.where(lanes == 0, jnp.full((L,), fs_code, jnp.int32), mrow)
    mrow = jnp.where(lanes == 1, jnp.full((L,), ls_code, jnp.int32), mrow)
    mrow = jnp.where(lanes == 2, jnp.full((L,), first_seg, jnp.int32), mrow)
    mrow = jnp.where(lanes == 3, jnp.full((L,), last_seg, jnp.int32), mrow)
    meta_v[...] = mrow
    pltpu.sync_copy(meta_v, meta_out.at[pl.ds(wid * L, L)])


@functools.partial(
    pl.kernel,
    out_type=jax.ShapeDtypeStruct((G * D,), jnp.float32),
    mesh=_mesh(),
    compiler_params=_params,
    scratch_types=[
        pltpu.VMEM((NW * L,), jnp.int32),
        pltpu.VMEM((SEG_PER_W * D,), jnp.float32),
        pltpu.VMEM((DROW,), jnp.float32),
        pltpu.VMEM((DROW,), jnp.float32),
        pltpu.SemaphoreType.DMA,
    ],
)
def _combine(out1_h, slab_h, meta_h, out_h, meta_v, obf, sum_v, tmp_v, sem_r):
    cid = lax.axis_index("c")
    sid = lax.axis_index("s")
    wid = sid * NC + cid
    sbase = wid * SEG_PER_W

    pltpu.async_copy(out1_h.at[pl.ds(sbase * D, SEG_PER_W * D)], obf, sem_r)
    pltpu.sync_copy(meta_h, meta_v)
    pltpu.make_async_copy(out1_h.at[pl.ds(sbase * D, SEG_PER_W * D)], obf,
                          sem_r).wait()

    zero = jnp.zeros((L,), jnp.float32)

    def _seg(g_local, _):
        g = sbase + g_local
        for j in range(NJ + 1):
            sum_v[pl.ds(j * L, L)] = zero

        def _scan(w2, carry):
            found, covered = carry
            mrow = meta_v[pl.ds(w2 * L, L)]
            fs = mrow[0]
            ls = mrow[1]
            f0 = mrow[2]
            l0 = mrow[3]
            covered = covered | ((f0 <= g) & (g <= l0))
            isfs = fs == g
            isls = ls == g

            @pl.when(isfs)
            def _():
                pltpu.sync_copy(slab_h.at[pl.ds(w2 * 2 * DROW, DROW)], tmp_v)
                for j in range(NJ + 1):
                    plsc.addupdate(sum_v.at[pl.ds(j * L, L)],
                                   tmp_v[pl.ds(j * L, L)])

            @pl.when(isls & jnp.logical_not(isfs))
            def _():
                pltpu.sync_copy(slab_h.at[pl.ds((w2 * 2 + 1) * DROW, DROW)],
                                tmp_v)
                for j in range(NJ + 1):
                    plsc.addupdate(sum_v.at[pl.ds(j * L, L)],
                                   tmp_v[pl.ds(j * L, L)])

            found = found | isfs | isls
            return found, covered

        found, covered = lax.fori_loop(
            0, NW, _scan, (jnp.bool_(False), jnp.bool_(False)))

        @pl.when(found)
        def _():
            scale = 1.0 / sum_v[pl.ds(D, L)]
            for j in range(NJ):
                obf[pl.ds(g_local * D + j * L, L)] = \
                    sum_v[pl.ds(j * L, L)] * scale

        @pl.when(jnp.logical_not(found | covered))
        def _():
            for j in range(NJ):
                obf[pl.ds(g_local * D + j * L, L)] = zero

        return 0

    lax.fori_loop(0, SEG_PER_W, _seg, 0)
    pltpu.sync_copy(obf, out_h.at[pl.ds(sbase * D, SEG_PER_W * D)])


def kernel(reference, attr, attention, batch_index):
    del reference
    att = attention.reshape((N,))
    bi = batch_index.astype(jnp.int32)
    out1, slab, meta = _pool(attr, att, bi)
    return _combine(out1, slab, meta).reshape(G, D)


# speedup vs baseline: 1.7139x; 1.7139x over previous
import functools

import jax
import jax.numpy as jnp
from jax import lax
from jax.experimental import pallas as pl
from jax.experimental.pallas import tpu as pltpu
from jax.experimental.pallas import tpu_sc as plsc

N = 100000
D = 512
G = 256
NC = 2
NS = 16
L = 16
NW = NC * NS
NJ = D // L
DROW = D + L

CH = L
NCHUNK = N // L
BIGW = (NCHUNK // 2) % NW
CPW = NCHUNK // 2 // NW
BIG_ROWS = (CPW + 1) * CH
SMALL_ROWS = CPW * CH
BIG_BASE = BIGW * BIG_ROWS

SEG_PER_W = G // NW

_SPLAT_DNUMS = lax.GatherDimensionNumbers(
    offset_dims=(), collapsed_slice_dims=(0,), start_index_map=(0,))


def _vgather(vec, idx):
    return lax.gather(vec, idx.reshape(L, 1), _SPLAT_DNUMS, slice_sizes=(1,),
                      mode=lax.GatherScatterMode.PROMISE_IN_BOUNDS)


def _splat(vec, lane):
    return _vgather(vec, jnp.full((L,), lane, jnp.int32))


_mesh = functools.partial(
    plsc.VectorSubcoreMesh,
    core_axis_name="c", subcore_axis_name="s", num_cores=NC, num_subcores=NS)

_params = pltpu.CompilerParams(needs_layout_passes=False)


@functools.partial(
    pl.kernel,
    out_type=[
        jax.ShapeDtypeStruct((G * D,), jnp.float32),
        jax.ShapeDtypeStruct((NW * 2 * DROW,), jnp.float32),
        jax.ShapeDtypeStruct((NW * L,), jnp.int32),
    ],
    mesh=_mesh(),
    compiler_params=_params,
    scratch_types=[
        pltpu.VMEM((CH, D), jnp.float32),
        pltpu.VMEM((CH, D), jnp.float32),
        pltpu.VMEM((BIG_ROWS,), jnp.float32),
        pltpu.VMEM((BIG_ROWS + L,), jnp.int32),
        pltpu.VMEM((L,), jnp.int32),
        pltpu.VMEM((L,), jnp.int32),
        pltpu.VMEM((D,), jnp.float32),
        pltpu.VMEM((L,), jnp.float32),
        pltpu.VMEM((D,), jnp.float32),
        pltpu.VMEM((L,), jnp.int32),
        pltpu.SemaphoreType.DMA,
        pltpu.SemaphoreType.DMA,
    ],
)
def _pool(attr_h, att_h, bi_h, out1, slab_out, meta_out,
          rows_a, rows_b, att_v, bi_v, ptail_v, nhead_v, acc_v, accw_v, zrow_v,
          meta_v, sem_a, sem_b):
    cid = lax.axis_index("c")
    sid = lax.axis_index("s")
    wid = sid * NC + cid

    nch = jnp.where(wid < BIGW, CPW + 1, CPW)
    nrows = nch * CH
    wbase = jnp.where(wid < BIGW, wid * BIG_ROWS,
                      BIG_BASE + (wid - BIGW) * SMALL_ROWS)

    zero = jnp.zeros((L,), jnp.float32)
    for j in range(NJ):
        acc_v[pl.ds(j * L, L)] = zero
        zrow_v[pl.ds(j * L, L)] = zero
    accw_v[...] = zero

    ptail_v[...] = jnp.full((L,), -1, jnp.int32)
    nhead_v[...] = jnp.full((L,), -1, jnp.int32)

    @pl.when(wid > 0)
    def _():
        pltpu.sync_copy(bi_h.at[pl.ds(wbase - L, L)], ptail_v)

    @pl.when(wid < NW - 1)
    def _():
        pltpu.sync_copy(bi_h.at[pl.ds(wbase + nrows, L)], nhead_v)

    @pl.when(wid < BIGW)
    def _():
        pltpu.sync_copy(att_h.at[pl.ds(wbase, BIG_ROWS)],
                        att_v.at[pl.ds(0, BIG_ROWS)])
        pltpu.sync_copy(bi_h.at[pl.ds(wbase, BIG_ROWS)],
                        bi_v.at[pl.ds(0, BIG_ROWS)])

    @pl.when(wid >= BIGW)
    def _():
        pltpu.sync_copy(att_h.at[pl.ds(wbase, SMALL_ROWS)],
                        att_v.at[pl.ds(0, SMALL_ROWS)])
        pltpu.sync_copy(bi_h.at[pl.ds(wbase, SMALL_ROWS)],
                        bi_v.at[pl.ds(0, SMALL_ROWS)])

    first_seg = bi_v[pl.ds(0, L)][0]
    last_seg = bi_v[pl.ds(nrows - L, L)][L - 1]
    first_shared = ptail_v[pl.ds(0, L)][L - 1] == first_seg
    last_shared = nhead_v[pl.ds(0, L)][0] == last_seg

    def _flush(seg):
        fs = (seg == first_seg) & first_shared
        ls = (seg == last_seg) & last_shared

        @pl.when(fs)
        def _():
            pltpu.sync_copy(acc_v, slab_out.at[pl.ds(wid * 2 * DROW, D)])
            pltpu.sync_copy(accw_v, slab_out.at[pl.ds(wid * 2 * DROW + D, L)])

        @pl.when(ls)
        def _():
            pltpu.sync_copy(acc_v,
                            slab_out.at[pl.ds((wid * 2 + 1) * DROW, D)])
            pltpu.sync_copy(accw_v,
                            slab_out.at[pl.ds((wid * 2 + 1) * DROW + D, L)])

        @pl.when(jnp.logical_not(fs | ls))
        def _():
            scale = 1.0 / accw_v[pl.ds(0, L)]
            for j in range(NJ):
                acc_v[pl.ds(j * L, L)] = acc_v[pl.ds(j * L, L)] * scale
            pltpu.sync_copy(acc_v, out1.at[pl.ds(seg * D, D)])

        for j in range(NJ):
            acc_v[pl.ds(j * L, L)] = zero
        accw_v[...] = zero

    lanes = lax.iota(jnp.int32, L)
    shift_idx = jnp.maximum(lanes - 1, 0)

    def _process(buf, rbase, lbase, prev):
        biv = bi_v[pl.ds(lbase, L)]
        wv = jnp.exp(att_v[pl.ds(lbase, L)])
        shifted = _vgather(biv, shift_idx)
        shifted = jnp.where(lanes == 0, jnp.full((L,), prev, jnp.int32), shifted)
        has_boundary = jnp.any(shifted != biv)

        @pl.when(jnp.logical_not(has_boundary))
        def _():
            wsum = _splat(wv, 0)
            for r in range(1, L):
                wsum = wsum + _splat(wv, r)
            plsc.addupdate(accw_v.at[pl.ds(0, L)], wsum)

            def tranche(t, _):
                jb = t * 8 * L
                w = _splat(wv, 0)
                regs = [buf[rbase, pl.ds(jb + j * L, L)] * w for j in range(8)]
                for r in range(1, L):
                    w = _splat(wv, r)
                    for j in range(8):
                        regs[j] = regs[j] + buf[rbase + r, pl.ds(jb + j * L, L)] * w
                for j in range(8):
                    plsc.addupdate(acc_v.at[pl.ds(jb + j * L, L)], regs[j])
                return 0
            lax.fori_loop(0, NJ // 8, tranche, 0)

        @pl.when(has_boundary)
        def _():
            def lane_body(li, prevl):
                s = bi_v[pl.ds(lbase + li, L)][0]

                @pl.when(s != prevl)
                def _():
                    _flush(prevl)

                    def gap_body(z):
                        pltpu.sync_copy(zrow_v, out1.at[pl.ds(z * D, D)])
                        return z + 1
                    lax.while_loop(lambda z: z < s, gap_body, prevl + 1)

                w = _vgather(wv, jnp.full((L,), li, jnp.int32))
                plsc.addupdate(accw_v.at[pl.ds(0, L)], w)
                for j in range(NJ):
                    plsc.addupdate(acc_v.at[pl.ds(j * L, L)],
                                   buf[rbase + li, pl.ds(j * L, L)] * w)
                return s
            lax.fori_loop(0, L, lane_body, prev)

        return biv[L - 1]

    pltpu.async_copy(attr_h.at[pl.ds(wbase, CH)], rows_a, sem_a)
    pltpu.async_copy(attr_h.at[pl.ds(wbase + CH, CH)], rows_b, sem_b)

    def _chunk(buf, c, prev):
        for gi in range(CH // L):
            prev = _process(buf, gi * L, c * CH + gi * L, prev)
        return prev

    def _pair(p, prev):
        c0 = 2 * p
        base0 = wbase + c0 * CH
        base1 = base0 + CH
        pltpu.make_async_copy(attr_h.at[pl.ds(base0, CH)], rows_a, sem_a).wait()
        prev = _chunk(rows_a, c0, prev)

        @pl.when(c0 + 2 < nch)
        def _():
            pltpu.async_copy(attr_h.at[pl.ds(base0 + 2 * CH, CH)], rows_a,
                             sem_a)

        pltpu.make_async_copy(attr_h.at[pl.ds(base1, CH)], rows_b, sem_b).wait()
        prev = _chunk(rows_b, c0 + 1, prev)

        @pl.when(c0 + 3 < nch)
        def _():
            pltpu.async_copy(attr_h.at[pl.ds(base1 + 2 * CH, CH)], rows_b,
                             sem_b)

        return prev

    prev = lax.fori_loop(0, nch // 2, _pair, first_seg)

    @pl.when(nch % 2 == 1)
    def _():
        c_t = nch - 1
        pltpu.make_async_copy(attr_h.at[pl.ds(wbase + c_t * CH, CH)], rows_a,
                              sem_a).wait()
        prev_t = _chunk(rows_a, c_t, prev)
        _flush(prev_t)

    @pl.when(nch % 2 == 0)
    def _():
        _flush(prev)

    fs_code = jnp.where(first_shared, first_seg, -1)
    ls_code = jnp.where(last_shared, last_seg, -1)
    mrow = jnp.full((L,), -1, jnp.int32)
    mrow = jnp.where(lanes == 0, jnp.full((L,), fs_code, jnp.int32), mrow)
    mrow = jnp.where(lanes == 1, jnp.full((L,), ls_code, jnp.int32), mrow)
    mrow = jnp.where(lanes == 2, jnp.full((L,), first_seg, jnp.int32), mrow)
    mrow = jnp.where(lanes == 3, jnp.full((L,), last_seg, jnp.int32), mrow)
    meta_v[...] = mrow
    pltpu.sync_copy(meta_v, meta_out.at[pl.ds(wid * L, L)])


@functools.partial(
    pl.kernel,
    out_type=jax.ShapeDtypeStruct((G * D,), jnp.float32),
    mesh=_mesh(),
    compiler_params=_params,
    scratch_types=[
        pltpu.VMEM((NW * L,), jnp.int32),
        pltpu.VMEM((SEG_PER_W * D,), jnp.float32),
        pltpu.VMEM((DROW,), jnp.float32),
        pltpu.VMEM((DROW,), jnp.float32),
        pltpu.SemaphoreType.DMA,
    ],
)
def _combine(out1_h, slab_h, meta_h, out_h, meta_v, obf, sum_v, tmp_v, sem_r):
    cid = lax.axis_index("c")
    sid = lax.axis_index("s")
    wid = sid * NC + cid
    sbase = wid * SEG_PER_W

    pltpu.async_copy(out1_h.at[pl.ds(sbase * D, SEG_PER_W * D)], obf, sem_r)
    pltpu.sync_copy(meta_h, meta_v)
    pltpu.make_async_copy(out1_h.at[pl.ds(sbase * D, SEG_PER_W * D)], obf,
                          sem_r).wait()

    zero = jnp.zeros((L,), jnp.float32)

    def _seg(g_local, _):
        g = sbase + g_local
        for j in range(NJ + 1):
            sum_v[pl.ds(j * L, L)] = zero

        def _scan(w2, carry):
            found, covered = carry
            mrow = meta_v[pl.ds(w2 * L, L)]
            fs = mrow[0]
            ls = mrow[1]
            f0 = mrow[2]
            l0 = mrow[3]
            covered = covered | ((f0 <= g) & (g <= l0))
            isfs = fs == g
            isls = ls == g

            @pl.when(isfs)
            def _():
                pltpu.sync_copy(slab_h.at[pl.ds(w2 * 2 * DROW, DROW)], tmp_v)
                for j in range(NJ + 1):
                    plsc.addupdate(sum_v.at[pl.ds(j * L, L)],
                                   tmp_v[pl.ds(j * L, L)])

            @pl.when(isls & jnp.logical_not(isfs))
            def _():
                pltpu.sync_copy(slab_h.at[pl.ds((w2 * 2 + 1) * DROW, DROW)],
                                tmp_v)
                for j in range(NJ + 1):
                    plsc.addupdate(sum_v.at[pl.ds(j * L, L)],
                                   tmp_v[pl.ds(j * L, L)])

            found = found | isfs | isls
            return found, covered

        found, covered = lax.fori_loop(
            0, NW, _scan, (jnp.bool_(False), jnp.bool_(False)))

        @pl.when(found)
        def _():
            scale = 1.0 / sum_v[pl.ds(D, L)]
            for j in range(NJ):
                obf[pl.ds(g_local * D + j * L, L)] = \
                    sum_v[pl.ds(j * L, L)] * scale

        @pl.when(jnp.logical_not(found | covered))
        def _():
            for j in range(NJ):
                obf[pl.ds(g_local * D + j * L, L)] = zero

        return 0

    lax.fori_loop(0, SEG_PER_W, _seg, 0)
    pltpu.sync_copy(obf, out_h.at[pl.ds(sbase * D, SEG_PER_W * D)])


def kernel(reference, attr, attention, batch_index):
    del reference
    att = attention.reshape((N,))
    bi = batch_index.astype(jnp.int32)
    out1, slab, meta = _pool(attr, att, bi)
    return _combine(out1, slab, meta).reshape(G, D)
